# Initial kernel scaffold; baseline (speedup 1.0000x reference)
import jax
import jax.numpy as jnp
from jax.experimental import pallas as pl


def kernel(q, kv, edge_index, W_k, b_k, W_v, b_v, W_o, b_o):
    raise NotImplementedError("write your pallas kernel here")


# speedup vs baseline: 14.9316x; 14.9316x over previous
import functools

import jax
import jax.numpy as jnp
import numpy as np
from jax import lax
from jax.experimental import pallas as pl
from jax.experimental.pallas import tpu as pltpu
from jax.experimental.pallas import tpu_sc as plsc

DIM = 128
HEADS = 8
HDIM = 16
N = 10000
E = 320000

NC = 2
NS = 16
HC = DIM // NC
HH = HEADS // NC
EPW = E // NS
CB = 80
NCHUNK = EPW // CB
NGRP = CB // 16
RPS = 624
DW = 8

_f32 = jnp.float32
_i32 = jnp.int32


def _proj_body(kv_ref, wk_ref, bk_ref, wv_ref, bv_ref, kv_out):
    x = kv_ref[...]
    dn = (((1,), (1,)), ((), ()))
    k = lax.dot_general(x, wk_ref[...], dn,
                        preferred_element_type=_f32) + bk_ref[...]
    v = lax.dot_general(x, wv_ref[...], dn,
                        preferred_element_type=_f32) + bv_ref[...]
    kv_out[0] = jnp.concatenate([k[:, :HC], v[:, :HC]], axis=1)
    kv_out[1] = jnp.concatenate([k[:, HC:], v[:, HC:]], axis=1)


def _project(kv, W_k, b_k, W_v, b_v):
    blk = 1000
    grid = N // blk
    return pl.pallas_call(
        _proj_body,
        grid=(grid,),
        in_specs=[
            pl.BlockSpec((blk, DIM), lambda i: (i, 0)),
            pl.BlockSpec((DIM, DIM), lambda i: (0, 0)),
            pl.BlockSpec((1, DIM), lambda i: (0, 0)),
            pl.BlockSpec((DIM, DIM), lambda i: (0, 0)),
            pl.BlockSpec((1, DIM), lambda i: (0, 0)),
        ],
        out_specs=pl.BlockSpec((NC, blk, DIM), lambda i: (0, i, 0)),
        out_shape=jax.ShapeDtypeStruct((NC, N, DIM), _f32),
    )(kv, W_k, b_k.reshape(1, DIM), W_v, b_v.reshape(1, DIM))


def _edge_body(kvt, qt, src2, dst2, agg_out, den_out,
               src_b, dst_b, kvbuf, qbuf, mbuf, exbuf,
               agg_sh, den_sh, gsem0, gsem1):
    cid = lax.axis_index("c")
    sid = lax.axis_index("s")

    zf = jnp.zeros((16,), _f32)
    lane = lax.iota(_i32, 16)

    def _zrow(r, _):
        for d in range(HC // 16):
            mbuf[0, r, pl.ds(d * 16, 16)] = zf
        exbuf[0, r, pl.ds(0, 16)] = zf
        exbuf[1, r, pl.ds(0, 16)] = zf
        return 0
    lax.fori_loop(0, CB, _zrow, 0)

    base = sid * RPS
    for i in range(RPS // CB):
        pltpu.sync_copy(mbuf.at[0], agg_sh.at[pl.ds(base + i * CB, CB)])
        pltpu.sync_copy(exbuf.at[0], den_sh.at[pl.ds(base + i * CB, CB)])
    rem = RPS - (RPS // CB) * CB
    pltpu.sync_copy(mbuf.at[0, pl.ds(0, rem)],
                    agg_sh.at[pl.ds(base + RPS - rem, rem)])
    pltpu.sync_copy(exbuf.at[0, pl.ds(0, rem)],
                    den_sh.at[pl.ds(base + RPS - rem, rem)])
    tail = N - NS * RPS
    @pl.when(sid == NS - 1)
    def _ztail():
        pltpu.sync_copy(mbuf.at[0, pl.ds(0, tail)],
                        agg_sh.at[pl.ds(N - tail, tail)])
        pltpu.sync_copy(exbuf.at[0, pl.ds(0, tail)],
                        den_sh.at[pl.ds(N - tail, tail)])
    plsc.subcore_barrier()

    pltpu.sync_copy(src2.at[sid], src_b)
    pltpu.sync_copy(dst2.at[sid], dst_b)

    kvtab = kvt.at[cid]
    qtab = qt.at[cid]
    gsems = (gsem0, gsem1)

    def g_issue(c, p):
        pltpu.async_copy(kvtab.at[src_b.at[c]], kvbuf.at[p], gsems[p])
        pltpu.async_copy(qtab.at[dst_b.at[c]], qbuf.at[p], gsems[p])

    def g_wait(c, p):
        pltpu.make_async_copy(kvtab.at[src_b.at[c]], kvbuf.at[p],
                              gsems[p]).wait()
        pltpu.make_async_copy(qtab.at[dst_b.at[c]], qbuf.at[p],
                              gsems[p]).wait()

    def s_issue(c, p):
        pltpu.sync_copy(mbuf.at[p], agg_sh.at[dst_b.at[c]], add=True)
        pltpu.sync_copy(exbuf.at[p], den_sh.at[dst_b.at[c]], add=True)

    def compute(p):
        def _group(g, _):
            erow = g * 16 + lane
            ccol = jnp.zeros((16,), _i32)
            exs = []
            for h in range(HH):
                part = [zf, zf, zf, zf]
                for _d in range(HDIM):
                    kvv = plsc.load_gather(kvbuf.at[p], [erow, ccol])
                    qvv = plsc.load_gather(qbuf.at[p], [erow, ccol])
                    part[_d % 4] = part[_d % 4] + kvv * qvv
                    ccol = ccol + 1
                s = (part[0] + part[1]) + (part[2] + part[3])
                ex = jnp.exp(s * 0.25)
                exs.append(ex)
                plsc.store_scatter(exbuf.at[p],
                                   [erow, jnp.full((16,), h, _i32)], ex)
            mcol = jnp.zeros((16,), _i32)
            vcol = jnp.full((16,), HC, _i32)
            for h in range(HH):
                for _d in range(HDIM):
                    vv = plsc.load_gather(kvbuf.at[p], [erow, vcol])
                    plsc.store_scatter(mbuf.at[p], [erow, mcol], vv * exs[h])
                    mcol = mcol + 1
                    vcol = vcol + 1
            return 0
        lax.fori_loop(0, NGRP, _group, 0)

    g_issue(0, 0)

    def _pair(i, _):
        cb2 = 2 * i
        for p in range(2):
            c = cb2 + p
            @pl.when(c < NCHUNK - 1)
            def _prefetch():
                g_issue(c + 1, 1 - p)
            g_wait(c, p)
            compute(p)
            s_issue(c, p)
        return 0
    lax.fori_loop(0, NCHUNK // 2, _pair, 0)

    plsc.subcore_barrier()
    pltpu.sync_copy(agg_sh.at[pl.ds(base, RPS)],
                    agg_out.at[cid, pl.ds(base, RPS)])
    pltpu.sync_copy(den_sh.at[pl.ds(base, RPS)],
                    den_out.at[cid, pl.ds(base, RPS)])
    @pl.when(sid == NS - 1)
    def _ptail():
        pltpu.sync_copy(agg_sh.at[pl.ds(N - tail, tail)],
                        agg_out.at[cid, pl.ds(N - tail, tail)])
        pltpu.sync_copy(den_sh.at[pl.ds(N - tail, tail)],
                        den_out.at[cid, pl.ds(N - tail, tail)])


_edge_kernel = functools.partial(
    pl.kernel,
    out_type=(
        jax.ShapeDtypeStruct((NC, N, HC), _f32),
        jax.ShapeDtypeStruct((NC, N, DW), _f32),
    ),
    mesh=plsc.VectorSubcoreMesh(core_axis_name="c", subcore_axis_name="s",
                                num_cores=NC, num_subcores=NS),
    compiler_params=pltpu.CompilerParams(needs_layout_passes=False,
                                         use_tc_tiling_on_sc=False),
    scratch_types=[
        pltpu.VMEM((NCHUNK, CB), _i32),
        pltpu.VMEM((NCHUNK, CB), _i32),
        pltpu.VMEM((2, CB, DIM), _f32),
        pltpu.VMEM((2, CB, HC), _f32),
        pltpu.VMEM((2, CB, HC), _f32),
        pltpu.VMEM((2, CB, DW), _f32),
        pltpu.VMEM_SHARED((N, HC), _f32),
        pltpu.VMEM_SHARED((N, DW), _f32),
        pltpu.SemaphoreType.DMA,
        pltpu.SemaphoreType.DMA,
    ],
)(_edge_body)


_REP = np.zeros((DW, HC), np.float32)
for _h in range(HH):
    _REP[_h, _h * HDIM:(_h + 1) * HDIM] = 1.0


def _comb_body(agg_ref, den_ref, wo_ref, bo_ref, rep_ref, out_ref):
    dn10 = (((1,), (0,)), ((), ()))
    dn11 = (((1,), (1,)), ((), ()))
    acc = None
    for c in range(NC):
        a = agg_ref[c]
        drep = lax.dot_general(den_ref[c], rep_ref[...], dn10,
                               preferred_element_type=_f32)
        r = a / (drep + 1e-16)
        part = lax.dot_general(r, wo_ref[:, c * HC:(c + 1) * HC], dn11,
                               preferred_element_type=_f32)
        acc = part if acc is None else acc + part
    out_ref[...] = acc + bo_ref[...]


def _combine(agg2, den2, W_o, b_o):
    blk = 1000
    grid = N // blk
    return pl.pallas_call(
        _comb_body,
        grid=(grid,),
        in_specs=[
            pl.BlockSpec((NC, blk, HC), lambda i: (0, i, 0)),
            pl.BlockSpec((NC, blk, DW), lambda i: (0, i, 0)),
            pl.BlockSpec((DIM, DIM), lambda i: (0, 0)),
            pl.BlockSpec((1, DIM), lambda i: (0, 0)),
            pl.BlockSpec((DW, HC), lambda i: (0, 0)),
        ],
        out_specs=pl.BlockSpec((blk, DIM), lambda i: (i, 0)),
        out_shape=jax.ShapeDtypeStruct((N, DIM), _f32),
    )(agg2, den2, W_o, b_o.reshape(1, DIM), jnp.asarray(_REP))


def kernel(q, kv, edge_index, W_k, b_k, W_v, b_v, W_o, b_o):
    KV2 = _project(kv, W_k, b_k, W_v, b_v)
    q2 = jnp.stack([q[:, :HC], q[:, HC:]])
    src2 = edge_index[0].reshape(NS, NCHUNK, CB)
    dst2 = edge_index[1].reshape(NS, NCHUNK, CB)
    agg2, den2 = _edge_kernel(KV2, q2, src2, dst2)
    return _combine(agg2, den2, W_o, b_o)
